# baseline (speedup 1.0000x reference)
import functools

import jax
import jax.numpy as jnp
from jax import lax
from jax.experimental import pallas as pl
from jax.experimental.pallas import tpu as pltpu
from jax.experimental.pallas import tpu_sc as plsc

NC = 2
NS = 16
NW = NC * NS
K = 80
NBUF = 5
DEG_W = 16

@functools.cache
def _mesh():
    return plsc.VectorSubcoreMesh(core_axis_name="c", subcore_axis_name="s")


_SC_PARAMS = pltpu.CompilerParams(
    use_tc_tiling_on_sc=False,
    skip_device_barrier=True,
    disable_semaphore_checks=True,
)


def _zero_spmem(zbuf, acc_sh, sid, rows_per_sub, zr, width):
    @pl.loop(0, zr)
    def _(i):
        for j in range(width // 16):
            zbuf[i, pl.ds(j * 16, 16)] = jnp.zeros((16,), jnp.float32)

    for r in range(rows_per_sub // zr):
        pltpu.sync_copy(zbuf, acc_sh.at[pl.ds(sid * rows_per_sub + r * zr, zr)])


def _pad_rows(n):
    unit = NS * 8 * 16
    return -(-n // unit) * unit


def _make_deg_call(n, e):
    epw = e // NW
    chunks = epw // K
    npad = _pad_rows(n)
    rps = npad // NS
    zr = min(rps, 128)
    assert rps % zr == 0

    @functools.partial(
        pl.kernel,
        mesh=_mesh(),
        out_type=jax.ShapeDtypeStruct((NC, npad, DEG_W), jnp.float32),
        scratch_types=[
            pltpu.VMEM((chunks, K), jnp.int32),
            pltpu.VMEM((K, DEG_W), jnp.float32),
            pltpu.VMEM((zr, DEG_W), jnp.float32),
            pltpu.VMEM_SHARED((npad, DEG_W), jnp.float32),
            pltpu.SemaphoreType.DMA,
        ],
        compiler_params=_SC_PARAMS,
    )
    def deg_call(dst_hbm, out_hbm, idx_v, ones_v, zbuf, acc_sh, sem):
        cid = lax.axis_index("c")
        sid = lax.axis_index("s")
        wid = sid * NC + cid

        @pl.loop(0, K)
        def _(i):
            ones_v[i, pl.ds(0, 16)] = jnp.full((16,), 1.0, jnp.float32)

        pltpu.async_copy(dst_hbm.at[wid], idx_v, sem)
        _zero_spmem(zbuf, acc_sh, sid, rps, zr, DEG_W)
        pltpu.make_async_copy(dst_hbm.at[wid], idx_v, sem).wait()
        plsc.subcore_barrier()

        @pl.loop(0, chunks)
        def _(c):
            pltpu.sync_copy(ones_v, acc_sh.at[idx_v.at[c]], add=True)

        plsc.subcore_barrier()
        pltpu.sync_copy(
            acc_sh.at[pl.ds(sid * rps, rps)],
            out_hbm.at[cid, pl.ds(sid * rps, rps)],
        )

    return deg_call


def _make_edge_call(n, e, hid):
    epw = e // NW
    chunks = epw // K
    npad = _pad_rows(n)
    rps = npad // NS
    zr = min(rps, 128)
    assert rps % zr == 0

    @functools.partial(
        pl.kernel,
        mesh=_mesh(),
        out_type=jax.ShapeDtypeStruct((NC, npad, hid), jnp.float32),
        scratch_types=[
            pltpu.VMEM((chunks, K), jnp.int32),
            pltpu.VMEM((chunks, K), jnp.int32),
            [pltpu.VMEM((K, hid), jnp.float32) for _ in range(NBUF)],
            pltpu.VMEM((zr, hid), jnp.float32),
            pltpu.VMEM_SHARED((npad, hid), jnp.float32),
            [pltpu.SemaphoreType.DMA for _ in range(NBUF)],
            pltpu.SemaphoreType.DMA,
        ],
        compiler_params=_SC_PARAMS,
    )
    def edge_call(hs_hbm, src_hbm, dst_hbm, out_hbm,
                  sidx_v, didx_v, rows, zbuf, acc_sh, gsems, isem):
        cid = lax.axis_index("c")
        sid = lax.axis_index("s")
        wid = sid * NC + cid

        pltpu.async_copy(src_hbm.at[wid], sidx_v, isem)
        pltpu.async_copy(dst_hbm.at[wid], didx_v, isem)
        _zero_spmem(zbuf, acc_sh, sid, rps, zr, hid)
        pltpu.make_async_copy(src_hbm.at[wid], sidx_v, isem).wait()
        pltpu.make_async_copy(dst_hbm.at[wid], didx_v, isem).wait()

        for b in range(NBUF):
            pltpu.async_copy(hs_hbm.at[sidx_v.at[b]], rows[b], gsems[b])

        plsc.subcore_barrier()

        @pl.loop(0, chunks, step=NBUF)
        def _(g):
            for b in range(NBUF):
                c = g + b
                pltpu.make_async_copy(
                    hs_hbm.at[sidx_v.at[c]], rows[b], gsems[b]).wait()
                pltpu.sync_copy(rows[b], acc_sh.at[didx_v.at[c]], add=True)
                nxt = c + NBUF

                @pl.when(nxt < chunks)
                def _():
                    pltpu.async_copy(
                        hs_hbm.at[sidx_v.at[nxt]], rows[b], gsems[b])

        plsc.subcore_barrier()
        pltpu.sync_copy(
            acc_sh.at[pl.ds(sid * rps, rps)],
            out_hbm.at[cid, pl.ds(sid * rps, rps)],
        )

    return edge_call


def _deg_dis(dacc_blk):
    deg = dacc_blk[0][:, 0:1] + dacc_blk[1][:, 0:1] + 1.0
    dis = lax.rsqrt(deg)
    return dis, dis * dis


def _k1_body(dacc_ref, x_ref, w1_ref, h_ref, hs_ref):
    dis, _ = _deg_dis(dacc_ref)
    h = jnp.dot(x_ref[...], w1_ref[...], preferred_element_type=jnp.float32)
    h_ref[...] = h
    hs_ref[...] = h * dis


def _k2_body(dacc_ref, acc_ref, h1_ref, w2_ref, b1_ref, h2_ref, h2s_ref):
    dis, dinv = _deg_dis(dacc_ref)
    z = dis * (acc_ref[0] + acc_ref[1]) + dinv * h1_ref[...] + b1_ref[...]
    a = jnp.where(z >= 0, z, 0.01 * z)
    h2 = jnp.dot(a, w2_ref[...], preferred_element_type=jnp.float32)
    h2_ref[...] = h2
    h2s_ref[...] = h2 * dis


def _k3_body(dacc_ref, acc_ref, h2_ref, b2_ref, out_ref):
    dis, dinv = _deg_dis(dacc_ref)
    z = dis * (acc_ref[0] + acc_ref[1]) + dinv * h2_ref[...] + b2_ref[...]
    out_ref[...] = jnp.where(z >= 0, z, 0.01 * z)


def kernel(x, edge_index, W1, b1, W2, b2):
    n, in_ch = x.shape
    e = edge_index.shape[1]
    hid = W1.shape[1]
    rb = 1000
    grid = (n // rb,)

    epw = e // NW
    chunks = epw // K
    assert epw % K == 0 and chunks % NBUF == 0
    src = edge_index[0].astype(jnp.int32).reshape(NW, chunks, K)
    dst = edge_index[1].astype(jnp.int32).reshape(NW, chunks, K)

    deg_call = _make_deg_call(n, e)
    edge_call = _make_edge_call(n, e, hid)

    dacc = deg_call(dst)

    dacc_spec = pl.BlockSpec((NC, rb, DEG_W), lambda i: (0, i, 0))
    acc_spec = pl.BlockSpec((NC, rb, hid), lambda i: (0, i, 0))
    row_spec = pl.BlockSpec((rb, hid), lambda i: (i, 0))
    bias_spec = pl.BlockSpec((1, hid), lambda i: (0, 0))

    h1, h1s = pl.pallas_call(
        _k1_body,
        grid=grid,
        in_specs=[
            dacc_spec,
            pl.BlockSpec((rb, in_ch), lambda i: (i, 0)),
            pl.BlockSpec((in_ch, hid), lambda i: (0, 0)),
        ],
        out_specs=[row_spec, row_spec],
        out_shape=[
            jax.ShapeDtypeStruct((n, hid), jnp.float32),
            jax.ShapeDtypeStruct((n, hid), jnp.float32),
        ],
    )(dacc, x, W1)

    acc1 = edge_call(h1s, src, dst)

    h2, h2s = pl.pallas_call(
        _k2_body,
        grid=grid,
        in_specs=[
            dacc_spec,
            acc_spec,
            row_spec,
            pl.BlockSpec((hid, hid), lambda i: (0, 0)),
            bias_spec,
        ],
        out_specs=[row_spec, row_spec],
        out_shape=[
            jax.ShapeDtypeStruct((n, hid), jnp.float32),
            jax.ShapeDtypeStruct((n, hid), jnp.float32),
        ],
    )(dacc, acc1, h1, W2, b1.reshape(1, hid))

    acc2 = edge_call(h2s, src, dst)

    out = pl.pallas_call(
        _k3_body,
        grid=grid,
        in_specs=[dacc_spec, acc_spec, row_spec, bias_spec],
        out_specs=row_spec,
        out_shape=jax.ShapeDtypeStruct((n, hid), jnp.float32),
    )(dacc, acc2, h2, b2.reshape(1, hid))

    return out


# speedup vs baseline: 1.0886x; 1.0886x over previous
import functools

import jax
import jax.numpy as jnp
from jax import lax
from jax.experimental import pallas as pl
from jax.experimental.pallas import tpu as pltpu
from jax.experimental.pallas import tpu_sc as plsc

NC = 2
NS = 16
NW = NC * NS
K = 80
NBUF = 5
DEG_W = 16

@functools.cache
def _mesh():
    return plsc.VectorSubcoreMesh(core_axis_name="c", subcore_axis_name="s")


_SC_PARAMS = pltpu.CompilerParams(use_tc_tiling_on_sc=False)


def _writeback(acc_sh, out_hbm, cid, sid, rps, width, sem):
    nb = rps // 8

    @pl.loop(0, nb)
    def _(g):
        pltpu.async_copy(
            acc_sh.at[pl.ds(sid * rps + g * 8, 8)],
            out_hbm.at[cid, sid * nb + g, :, pl.ds(0, width)],
            sem)

    @pl.loop(0, nb)
    def _(g):
        pltpu.make_async_copy(
            acc_sh.at[pl.ds(sid * rps + g * 8, 8)],
            out_hbm.at[cid, sid * nb + g, :, pl.ds(0, width)],
            sem).wait()


def _zero_spmem(zbuf, acc_sh, sid, rows_per_sub, zr, width):
    @pl.loop(0, zr)
    def _(i):
        for j in range(width // 16):
            zbuf[i, pl.ds(j * 16, 16)] = jnp.zeros((16,), jnp.float32)

    for r in range(rows_per_sub // zr):
        pltpu.sync_copy(zbuf, acc_sh.at[pl.ds(sid * rows_per_sub + r * zr, zr)])


def _pad_rows(n):
    unit = NS * 8 * 16
    return -(-n // unit) * unit


def _make_deg_call(n, e):
    epw = e // NW
    chunks = epw // K
    npad = _pad_rows(n)
    rps = npad // NS
    zr = min(rps, 128)
    assert rps % zr == 0

    @functools.partial(
        pl.kernel,
        mesh=_mesh(),
        out_type=jax.ShapeDtypeStruct((NC, npad // 8, 8, 128), jnp.float32),
        scratch_types=[
            pltpu.VMEM((chunks, K), jnp.int32),
            pltpu.VMEM((K, DEG_W), jnp.float32),
            pltpu.VMEM((zr, DEG_W), jnp.float32),
            pltpu.VMEM_SHARED((npad, DEG_W), jnp.float32),
            pltpu.SemaphoreType.DMA,
        ],
        compiler_params=_SC_PARAMS,
    )
    def deg_call(dst_hbm, out_hbm, idx_v, ones_v, zbuf, acc_sh, sem):
        cid = lax.axis_index("c")
        sid = lax.axis_index("s")
        wid = sid * NC + cid

        @pl.loop(0, K)
        def _(i):
            ones_v[i, pl.ds(0, 16)] = jnp.full((16,), 1.0, jnp.float32)

        pltpu.async_copy(dst_hbm.at[wid], idx_v, sem)
        _zero_spmem(zbuf, acc_sh, sid, rps, zr, DEG_W)
        pltpu.make_async_copy(dst_hbm.at[wid], idx_v, sem).wait()
        plsc.subcore_barrier()

        @pl.loop(0, chunks)
        def _(c):
            pltpu.sync_copy(ones_v, acc_sh.at[idx_v.at[c]], add=True)

        plsc.subcore_barrier()
        _writeback(acc_sh, out_hbm, cid, sid, rps, DEG_W, sem)

    return deg_call


def _make_edge_call(n, e, hid):
    epw = e // NW
    chunks = epw // K
    npad = _pad_rows(n)
    rps = npad // NS
    zr = min(rps, 128)
    assert rps % zr == 0

    @functools.partial(
        pl.kernel,
        mesh=_mesh(),
        out_type=jax.ShapeDtypeStruct((NC, npad // 8, 8, 128), jnp.float32),
        scratch_types=[
            pltpu.VMEM((chunks, K), jnp.int32),
            pltpu.VMEM((chunks, K), jnp.int32),
            [pltpu.VMEM((K, hid), jnp.float32) for _ in range(NBUF)],
            pltpu.VMEM((zr, hid), jnp.float32),
            pltpu.VMEM_SHARED((npad, hid), jnp.float32),
            [pltpu.SemaphoreType.DMA for _ in range(NBUF)],
            pltpu.SemaphoreType.DMA,
        ],
        compiler_params=_SC_PARAMS,
    )
    def edge_call(hs_hbm, src_hbm, dst_hbm, out_hbm,
                  sidx_v, didx_v, rows, zbuf, acc_sh, gsems, isem):
        cid = lax.axis_index("c")
        sid = lax.axis_index("s")
        wid = sid * NC + cid

        pltpu.async_copy(src_hbm.at[wid], sidx_v, isem)
        pltpu.async_copy(dst_hbm.at[wid], didx_v, isem)
        _zero_spmem(zbuf, acc_sh, sid, rps, zr, hid)
        pltpu.make_async_copy(src_hbm.at[wid], sidx_v, isem).wait()
        pltpu.make_async_copy(dst_hbm.at[wid], didx_v, isem).wait()

        for b in range(NBUF):
            pltpu.async_copy(hs_hbm.at[sidx_v.at[b]], rows[b], gsems[b])

        plsc.subcore_barrier()

        @pl.loop(0, chunks, step=NBUF)
        def _(g):
            for b in range(NBUF):
                c = g + b
                pltpu.make_async_copy(
                    hs_hbm.at[sidx_v.at[c]], rows[b], gsems[b]).wait()
                pltpu.sync_copy(rows[b], acc_sh.at[didx_v.at[c]], add=True)
                nxt = c + NBUF

                @pl.when(nxt < chunks)
                def _():
                    pltpu.async_copy(
                        hs_hbm.at[sidx_v.at[nxt]], rows[b], gsems[b])

        plsc.subcore_barrier()
        _writeback(acc_sh, out_hbm, cid, sid, rps, hid, isem)

    return edge_call


def _deg_dis(dacc_ref, rb):
    d = jnp.reshape(dacc_ref[:, :, :, 0:1], (2, rb, 1))
    deg = d[0] + d[1] + 1.0
    dis = lax.rsqrt(deg)
    return dis, dis * dis


def _unpack_acc(acc_ref, rb, hid):
    a = jnp.reshape(acc_ref[:, :, :, 0:hid], (2, rb, hid))
    return a[0] + a[1]


def _k1_body(dacc_ref, x_ref, w1_ref, h_ref, hs_ref):
    rb = x_ref.shape[0]
    dis, _ = _deg_dis(dacc_ref, rb)
    h = jnp.dot(x_ref[...], w1_ref[...], preferred_element_type=jnp.float32)
    h_ref[...] = h
    hs_ref[...] = h * dis


def _k2_body(dacc_ref, acc_ref, h1_ref, w2_ref, b1_ref, h2_ref, h2s_ref):
    rb, hid = h1_ref.shape
    dis, dinv = _deg_dis(dacc_ref, rb)
    z = dis * _unpack_acc(acc_ref, rb, hid) + dinv * h1_ref[...] + b1_ref[...]
    a = jnp.where(z >= 0, z, 0.01 * z)
    h2 = jnp.dot(a, w2_ref[...], preferred_element_type=jnp.float32)
    h2_ref[...] = h2
    h2s_ref[...] = h2 * dis


def _k3_body(dacc_ref, acc_ref, h2_ref, b2_ref, out_ref):
    rb, hid = h2_ref.shape
    dis, dinv = _deg_dis(dacc_ref, rb)
    z = dis * _unpack_acc(acc_ref, rb, hid) + dinv * h2_ref[...] + b2_ref[...]
    out_ref[...] = jnp.where(z >= 0, z, 0.01 * z)


def kernel(x, edge_index, W1, b1, W2, b2):
    n, in_ch = x.shape
    e = edge_index.shape[1]
    hid = W1.shape[1]
    npad = _pad_rows(n)
    rb = 1024
    grid = (npad // rb,)

    epw = e // NW
    chunks = epw // K
    assert epw % K == 0 and chunks % NBUF == 0
    src = edge_index[0].astype(jnp.int32).reshape(NW, chunks, K)
    dst = edge_index[1].astype(jnp.int32).reshape(NW, chunks, K)

    deg_call = _make_deg_call(n, e)
    edge_call = _make_edge_call(n, e, hid)

    dacc = deg_call(dst)

    sc_spec = pl.BlockSpec((NC, rb // 8, 8, 128), lambda i: (0, i, 0, 0))
    row_spec = pl.BlockSpec((rb, hid), lambda i: (i, 0))
    bias_spec = pl.BlockSpec((1, hid), lambda i: (0, 0))

    h1, h1s = pl.pallas_call(
        _k1_body,
        grid=grid,
        in_specs=[
            sc_spec,
            pl.BlockSpec((rb, in_ch), lambda i: (i, 0)),
            pl.BlockSpec((in_ch, hid), lambda i: (0, 0)),
        ],
        out_specs=[row_spec, row_spec],
        out_shape=[
            jax.ShapeDtypeStruct((npad, hid), jnp.float32),
            jax.ShapeDtypeStruct((npad, hid), jnp.float32),
        ],
    )(dacc, x, W1)

    acc1 = edge_call(h1s, src, dst)

    h2, h2s = pl.pallas_call(
        _k2_body,
        grid=grid,
        in_specs=[
            sc_spec,
            sc_spec,
            row_spec,
            pl.BlockSpec((hid, hid), lambda i: (0, 0)),
            bias_spec,
        ],
        out_specs=[row_spec, row_spec],
        out_shape=[
            jax.ShapeDtypeStruct((npad, hid), jnp.float32),
            jax.ShapeDtypeStruct((npad, hid), jnp.float32),
        ],
    )(dacc, acc1, h1, W2, b1.reshape(1, hid))

    acc2 = edge_call(h2s, src, dst)

    out = pl.pallas_call(
        _k3_body,
        grid=grid,
        in_specs=[sc_spec, sc_spec, row_spec, bias_spec],
        out_specs=row_spec,
        out_shape=jax.ShapeDtypeStruct((n, hid), jnp.float32),
    )(dacc, acc2, h2, b2.reshape(1, hid))

    return out


# speedup vs baseline: 1.1911x; 1.0942x over previous
import functools

import jax
import jax.numpy as jnp
from jax import lax
from jax.experimental import pallas as pl
from jax.experimental.pallas import tpu as pltpu
from jax.experimental.pallas import tpu_sc as plsc

NC = 2
NS = 16
NW = NC * NS
K = 80
NBUF = 5
DEG_W = 16

@functools.cache
def _mesh():
    return plsc.VectorSubcoreMesh(core_axis_name="c", subcore_axis_name="s")


_SC_PARAMS = pltpu.CompilerParams(use_tc_tiling_on_sc=False)


def _writeback(acc_sh, out_hbm, cid, sid, rps, width, sem):
    nb = rps // 8

    @pl.loop(0, nb)
    def _(g):
        pltpu.async_copy(
            acc_sh.at[pl.ds(sid * rps + g * 8, 8)],
            out_hbm.at[cid, sid * nb + g, :, pl.ds(0, width)],
            sem)

    @pl.loop(0, nb)
    def _(g):
        pltpu.make_async_copy(
            acc_sh.at[pl.ds(sid * rps + g * 8, 8)],
            out_hbm.at[cid, sid * nb + g, :, pl.ds(0, width)],
            sem).wait()


def _zero_spmem(zbuf, acc_sh, sid, rows_per_sub, zr, width):
    @pl.loop(0, zr)
    def _(i):
        for j in range(width // 16):
            zbuf[i, pl.ds(j * 16, 16)] = jnp.zeros((16,), jnp.float32)

    for r in range(rows_per_sub // zr):
        pltpu.sync_copy(zbuf, acc_sh.at[pl.ds(sid * rows_per_sub + r * zr, zr)])


def _pad_rows(n):
    unit = NS * 8 * 16
    return -(-n // unit) * unit


def _make_deg_call(n, e):
    epw = e // NW
    chunks = epw // K
    npad = _pad_rows(n)
    rps = npad // NS
    zr = min(rps, 128)
    assert rps % zr == 0

    @functools.partial(
        pl.kernel,
        mesh=_mesh(),
        out_type=jax.ShapeDtypeStruct((NC, npad // 8, 8, 128), jnp.float32),
        scratch_types=[
            pltpu.VMEM((chunks, K), jnp.int32),
            pltpu.VMEM((K, DEG_W), jnp.float32),
            pltpu.VMEM((zr, DEG_W), jnp.float32),
            pltpu.VMEM_SHARED((npad, DEG_W), jnp.float32),
            pltpu.SemaphoreType.DMA,
            pltpu.SemaphoreType.DMA,
        ],
        compiler_params=_SC_PARAMS,
    )
    def deg_call(ei_hbm, out_hbm, idx_v, ones_v, zbuf, acc_sh, sem, ssem):
        cid = lax.axis_index("c")
        sid = lax.axis_index("s")
        wid = sid * NC + cid

        @pl.loop(0, K)
        def _(i):
            ones_v[i, pl.ds(0, 16)] = jnp.full((16,), 1.0, jnp.float32)

        pltpu.async_copy(ei_hbm.at[1, wid], idx_v, sem)
        _zero_spmem(zbuf, acc_sh, sid, rps, zr, DEG_W)
        pltpu.make_async_copy(ei_hbm.at[1, wid], idx_v, sem).wait()
        plsc.subcore_barrier()

        @pl.loop(0, chunks)
        def _(c):
            pltpu.async_copy(ones_v, acc_sh.at[idx_v.at[c]], ssem, add=True)

        @pl.loop(0, chunks)
        def _(c):
            pltpu.make_async_copy(
                ones_v, acc_sh.at[idx_v.at[c]], ssem).wait()

        plsc.subcore_barrier()
        _writeback(acc_sh, out_hbm, cid, sid, rps, DEG_W, sem)

    return deg_call


def _make_edge_call(n, e, hid):
    epw = e // NW
    chunks = epw // K
    npad = _pad_rows(n)
    rps = npad // NS
    zr = min(rps, 128)
    assert rps % zr == 0

    @functools.partial(
        pl.kernel,
        mesh=_mesh(),
        out_type=jax.ShapeDtypeStruct((NC, npad // 8, 8, 128), jnp.float32),
        scratch_types=[
            pltpu.VMEM((chunks, K), jnp.int32),
            pltpu.VMEM((chunks, K), jnp.int32),
            [pltpu.VMEM((K, hid), jnp.float32) for _ in range(NBUF)],
            pltpu.VMEM((zr, hid), jnp.float32),
            pltpu.VMEM_SHARED((npad, hid), jnp.float32),
            [pltpu.SemaphoreType.DMA for _ in range(NBUF)],
            pltpu.SemaphoreType.DMA,
        ],
        compiler_params=_SC_PARAMS,
    )
    def edge_call(hs_hbm, ei_hbm, out_hbm,
                  sidx_v, didx_v, rows, zbuf, acc_sh, gsems, isem):
        cid = lax.axis_index("c")
        sid = lax.axis_index("s")
        wid = sid * NC + cid

        pltpu.async_copy(ei_hbm.at[0, wid], sidx_v, isem)
        pltpu.async_copy(ei_hbm.at[1, wid], didx_v, isem)
        _zero_spmem(zbuf, acc_sh, sid, rps, zr, hid)
        pltpu.make_async_copy(ei_hbm.at[0, wid], sidx_v, isem).wait()
        pltpu.make_async_copy(ei_hbm.at[1, wid], didx_v, isem).wait()

        for b in range(NBUF):
            pltpu.async_copy(hs_hbm.at[sidx_v.at[b]], rows[b], gsems[b])

        plsc.subcore_barrier()

        @pl.loop(0, chunks, step=NBUF)
        def _(g):
            for b in range(NBUF):
                c = g + b
                pltpu.make_async_copy(
                    hs_hbm.at[sidx_v.at[c]], rows[b], gsems[b]).wait()
                pltpu.sync_copy(rows[b], acc_sh.at[didx_v.at[c]], add=True)
                nxt = c + NBUF

                @pl.when(nxt < chunks)
                def _():
                    pltpu.async_copy(
                        hs_hbm.at[sidx_v.at[nxt]], rows[b], gsems[b])

        plsc.subcore_barrier()
        _writeback(acc_sh, out_hbm, cid, sid, rps, hid, isem)

    return edge_call


def _deg_dis(dacc_ref, rb):
    d = jnp.reshape(dacc_ref[:, :, :, 0:1], (2, rb, 1))
    deg = d[0] + d[1] + 1.0
    dis = lax.rsqrt(deg)
    return dis, dis * dis


def _unpack_acc(acc_ref, rb, hid):
    a = jnp.reshape(acc_ref[:, :, :, 0:hid], (2, rb, hid))
    return a[0] + a[1]


def _k1_body(dacc_ref, x_ref, w1_ref, h_ref, hs_ref):
    rb = x_ref.shape[0]
    dis, _ = _deg_dis(dacc_ref, rb)
    h = jnp.dot(x_ref[...], w1_ref[...], preferred_element_type=jnp.float32)
    h_ref[...] = h
    hs_ref[...] = h * dis


def _k2_body(dacc_ref, acc_ref, h1_ref, w2_ref, b1_ref, h2_ref, h2s_ref):
    rb, hid = h1_ref.shape
    dis, dinv = _deg_dis(dacc_ref, rb)
    z = dis * _unpack_acc(acc_ref, rb, hid) + dinv * h1_ref[...] + b1_ref[...]
    a = jnp.where(z >= 0, z, 0.01 * z)
    h2 = jnp.dot(a, w2_ref[...], preferred_element_type=jnp.float32)
    h2_ref[...] = h2
    h2s_ref[...] = h2 * dis


def _k3_body(dacc_ref, acc_ref, h2_ref, b2_ref, out_ref):
    rb, hid = h2_ref.shape
    dis, dinv = _deg_dis(dacc_ref, rb)
    z = dis * _unpack_acc(acc_ref, rb, hid) + dinv * h2_ref[...] + b2_ref[...]
    out_ref[...] = jnp.where(z >= 0, z, 0.01 * z)


def kernel(x, edge_index, W1, b1, W2, b2):
    n, in_ch = x.shape
    e = edge_index.shape[1]
    hid = W1.shape[1]
    npad = _pad_rows(n)
    rb = 1024
    grid = (npad // rb,)

    epw = e // NW
    chunks = epw // K
    assert epw % K == 0 and chunks % NBUF == 0
    ei = edge_index.astype(jnp.int32).reshape(2, NW, chunks, K)

    deg_call = _make_deg_call(n, e)
    edge_call = _make_edge_call(n, e, hid)

    dacc = deg_call(ei)

    deg_spec = pl.BlockSpec((NC, rb // 8, 8, 128), lambda i: (0, i, 0, 0))
    accs_spec = pl.BlockSpec((NC, rb // 8, 8, 128), lambda i: (0, i, 0, 0))
    row_spec = pl.BlockSpec((rb, hid), lambda i: (i, 0))
    bias_spec = pl.BlockSpec((1, hid), lambda i: (0, 0))

    h1, h1s = pl.pallas_call(
        _k1_body,
        grid=grid,
        in_specs=[
            deg_spec,
            pl.BlockSpec((rb, in_ch), lambda i: (i, 0)),
            pl.BlockSpec((in_ch, hid), lambda i: (0, 0)),
        ],
        out_specs=[row_spec, row_spec],
        out_shape=[
            jax.ShapeDtypeStruct((npad, hid), jnp.float32),
            jax.ShapeDtypeStruct((npad, hid), jnp.float32),
        ],
    )(dacc, x, W1)

    acc1 = edge_call(h1s, ei)

    h2, h2s = pl.pallas_call(
        _k2_body,
        grid=grid,
        in_specs=[
            deg_spec,
            accs_spec,
            row_spec,
            pl.BlockSpec((hid, hid), lambda i: (0, 0)),
            bias_spec,
        ],
        out_specs=[row_spec, row_spec],
        out_shape=[
            jax.ShapeDtypeStruct((npad, hid), jnp.float32),
            jax.ShapeDtypeStruct((npad, hid), jnp.float32),
        ],
    )(dacc, acc1, h1, W2, b1.reshape(1, hid))

    acc2 = edge_call(h2s, ei)

    out = pl.pallas_call(
        _k3_body,
        grid=grid,
        in_specs=[deg_spec, accs_spec, row_spec, bias_spec],
        out_specs=row_spec,
        out_shape=jax.ShapeDtypeStruct((n, hid), jnp.float32),
    )(dacc, acc2, h2, b2.reshape(1, hid))

    return out


# speedup vs baseline: 1.2139x; 1.0191x over previous
import functools

import jax
import jax.numpy as jnp
from jax import lax
from jax.experimental import pallas as pl
from jax.experimental.pallas import tpu as pltpu
from jax.experimental.pallas import tpu_sc as plsc

NC = 2
NS = 16
NW = NC * NS
K = 80
NBUF = 5
DEG_W = 16

@functools.cache
def _mesh():
    return plsc.VectorSubcoreMesh(core_axis_name="c", subcore_axis_name="s")


_SC_PARAMS = pltpu.CompilerParams(use_tc_tiling_on_sc=False)


def _writeback(acc_sh, out_hbm, cid, sid, rps, width, sem):
    nb = rps // 8

    @pl.loop(0, nb)
    def _(g):
        pltpu.async_copy(
            acc_sh.at[pl.ds(sid * rps + g * 8, 8)],
            out_hbm.at[cid, sid * nb + g, :, pl.ds(0, width)],
            sem)

    @pl.loop(0, nb)
    def _(g):
        pltpu.make_async_copy(
            acc_sh.at[pl.ds(sid * rps + g * 8, 8)],
            out_hbm.at[cid, sid * nb + g, :, pl.ds(0, width)],
            sem).wait()


def _zero_spmem(zbuf, acc_sh, sid, rows_per_sub, zr, width):
    @pl.loop(0, zr)
    def _(i):
        for j in range(width // 16):
            zbuf[i, pl.ds(j * 16, 16)] = jnp.zeros((16,), jnp.float32)

    for r in range(rows_per_sub // zr):
        pltpu.sync_copy(zbuf, acc_sh.at[pl.ds(sid * rows_per_sub + r * zr, zr)])


def _pad_rows(n):
    unit = NS * 8 * 16
    return -(-n // unit) * unit


def _make_deg_call(n, e):
    epw = e // NW
    chunks = epw // K
    npad = _pad_rows(n)
    rps = npad // NS
    zr = min(rps, 128)
    assert rps % zr == 0

    @functools.partial(
        pl.kernel,
        mesh=_mesh(),
        out_type=jax.ShapeDtypeStruct((NC, npad // 8, 8, 128), jnp.float32),
        scratch_types=[
            pltpu.VMEM((chunks, K), jnp.int32),
            pltpu.VMEM((K, DEG_W), jnp.float32),
            pltpu.VMEM((zr, DEG_W), jnp.float32),
            pltpu.VMEM_SHARED((npad, DEG_W), jnp.float32),
            pltpu.SemaphoreType.DMA,
            pltpu.SemaphoreType.DMA,
        ],
        compiler_params=_SC_PARAMS,
    )
    def deg_call(ei_hbm, out_hbm, idx_v, ones_v, zbuf, acc_sh, sem, ssem):
        cid = lax.axis_index("c")
        sid = lax.axis_index("s")
        wid = sid * NC + cid

        @pl.loop(0, K)
        def _(i):
            ones_v[i, pl.ds(0, 16)] = jnp.full((16,), 1.0, jnp.float32)

        pltpu.async_copy(ei_hbm.at[1, wid], idx_v, sem)
        _zero_spmem(zbuf, acc_sh, sid, rps, zr, DEG_W)
        pltpu.make_async_copy(ei_hbm.at[1, wid], idx_v, sem).wait()
        plsc.subcore_barrier()

        @pl.loop(0, chunks)
        def _(c):
            pltpu.async_copy(ones_v, acc_sh.at[idx_v.at[c]], ssem, add=True)

        @pl.loop(0, chunks)
        def _(c):
            pltpu.make_async_copy(
                ones_v, acc_sh.at[idx_v.at[c]], ssem).wait()

        plsc.subcore_barrier()
        _writeback(acc_sh, out_hbm, cid, sid, rps, DEG_W, sem)

    return deg_call


def _make_edge_call(n, e, hid):
    epw = e // NW
    chunks = epw // K
    npad = _pad_rows(n)
    rps = npad // NS
    zr = min(rps, 128)
    assert rps % zr == 0

    @functools.partial(
        pl.kernel,
        mesh=_mesh(),
        out_type=jax.ShapeDtypeStruct((NC, npad // 8, 8, 128), jnp.float32),
        scratch_types=[
            pltpu.VMEM((chunks, K), jnp.int32),
            pltpu.VMEM((chunks, K), jnp.int32),
            [pltpu.VMEM((K, hid), jnp.float32) for _ in range(NBUF)],
            pltpu.VMEM((zr, hid), jnp.float32),
            pltpu.VMEM_SHARED((npad, hid), jnp.float32),
            [pltpu.SemaphoreType.DMA for _ in range(NBUF)],
            pltpu.SemaphoreType.DMA,
        ],
        compiler_params=_SC_PARAMS,
    )
    def edge_call(hs_hbm, ei_hbm, out_hbm,
                  sidx_v, didx_v, rows, zbuf, acc_sh, gsems, isem):
        cid = lax.axis_index("c")
        sid = lax.axis_index("s")
        wid = sid * NC + cid

        pltpu.async_copy(ei_hbm.at[0, wid], sidx_v, isem)
        pltpu.async_copy(ei_hbm.at[1, wid], didx_v, isem)
        _zero_spmem(zbuf, acc_sh, sid, rps, zr, hid)
        pltpu.make_async_copy(ei_hbm.at[0, wid], sidx_v, isem).wait()
        pltpu.make_async_copy(ei_hbm.at[1, wid], didx_v, isem).wait()

        for b in range(NBUF):
            pltpu.async_copy(hs_hbm.at[sidx_v.at[b]], rows[b], gsems[b])

        plsc.subcore_barrier()

        @pl.loop(0, chunks, step=NBUF)
        def _(g):
            for b in range(NBUF):
                c = g + b
                pltpu.make_async_copy(
                    hs_hbm.at[sidx_v.at[c]], rows[b], gsems[b]).wait()
                pltpu.sync_copy(rows[b], acc_sh.at[didx_v.at[c]], add=True)
                nxt = c + NBUF

                @pl.when(nxt < chunks)
                def _():
                    pltpu.async_copy(
                        hs_hbm.at[sidx_v.at[nxt]], rows[b], gsems[b])

        plsc.subcore_barrier()
        _writeback(acc_sh, out_hbm, cid, sid, rps, hid, isem)

    return edge_call


def _deg_dis(dacc_ref, rb):
    d = jnp.reshape(dacc_ref[:, :, :, 0:1], (2, rb, 1))
    deg = d[0] + d[1] + 1.0
    return lax.rsqrt(deg)


def _unpack_acc(acc_ref, rb, hid):
    a = jnp.reshape(acc_ref[:, :, :, 0:hid], (2, rb, hid))
    return a[0] + a[1]


def _kmm_body(x_ref, w1_ref, h_ref):
    h_ref[...] = jnp.dot(x_ref[...], w1_ref[...],
                         preferred_element_type=jnp.float32)


def _ksc_body(dacc_ref, h_ref, hs_ref):
    dis = _deg_dis(dacc_ref, h_ref.shape[0])
    hs_ref[...] = h_ref[...] * dis


def _k2_body(dacc_ref, acc_ref, hs1_ref, w2_ref, b1_ref, h2s_ref):
    rb, hid = hs1_ref.shape
    dis = _deg_dis(dacc_ref, rb)
    z = dis * (_unpack_acc(acc_ref, rb, hid) + hs1_ref[...]) + b1_ref[...]
    a = jnp.where(z >= 0, z, 0.01 * z)
    h2 = jnp.dot(a, w2_ref[...], preferred_element_type=jnp.float32)
    h2s_ref[...] = h2 * dis


def _k3_body(dacc_ref, acc_ref, hs2_ref, b2_ref, out_ref):
    rb, hid = hs2_ref.shape
    dis = _deg_dis(dacc_ref, rb)
    z = dis * (_unpack_acc(acc_ref, rb, hid) + hs2_ref[...]) + b2_ref[...]
    out_ref[...] = jnp.where(z >= 0, z, 0.01 * z)


def kernel(x, edge_index, W1, b1, W2, b2):
    n, in_ch = x.shape
    e = edge_index.shape[1]
    hid = W1.shape[1]
    npad = _pad_rows(n)
    rb = 1024
    grid = (npad // rb,)

    epw = e // NW
    chunks = epw // K
    assert epw % K == 0 and chunks % NBUF == 0
    ei = edge_index.astype(jnp.int32).reshape(2, NW, chunks, K)

    deg_call = _make_deg_call(n, e)
    edge_call = _make_edge_call(n, e, hid)

    dacc = deg_call(ei)

    deg_spec = pl.BlockSpec((NC, rb // 8, 8, 128), lambda i: (0, i, 0, 0))
    accs_spec = pl.BlockSpec((NC, rb // 8, 8, 128), lambda i: (0, i, 0, 0))
    row_spec = pl.BlockSpec((rb, hid), lambda i: (i, 0))
    bias_spec = pl.BlockSpec((1, hid), lambda i: (0, 0))

    h1 = pl.pallas_call(
        _kmm_body,
        grid=grid,
        in_specs=[
            pl.BlockSpec((rb, in_ch), lambda i: (i, 0)),
            pl.BlockSpec((in_ch, hid), lambda i: (0, 0)),
        ],
        out_specs=row_spec,
        out_shape=jax.ShapeDtypeStruct((npad, hid), jnp.float32),
    )(x, W1)

    h1s = pl.pallas_call(
        _ksc_body,
        grid=grid,
        in_specs=[deg_spec, row_spec],
        out_specs=row_spec,
        out_shape=jax.ShapeDtypeStruct((npad, hid), jnp.float32),
    )(dacc, h1)

    acc1 = edge_call(h1s, ei)

    h2s = pl.pallas_call(
        _k2_body,
        grid=grid,
        in_specs=[
            deg_spec,
            accs_spec,
            row_spec,
            pl.BlockSpec((hid, hid), lambda i: (0, 0)),
            bias_spec,
        ],
        out_specs=row_spec,
        out_shape=jax.ShapeDtypeStruct((npad, hid), jnp.float32),
    )(dacc, acc1, h1s, W2, b1.reshape(1, hid))

    acc2 = edge_call(h2s, ei)

    out = pl.pallas_call(
        _k3_body,
        grid=grid,
        in_specs=[deg_spec, accs_spec, row_spec, bias_spec],
        out_specs=row_spec,
        out_shape=jax.ShapeDtypeStruct((n, hid), jnp.float32),
    )(dacc, acc2, h2s, b2.reshape(1, hid))

    return out


# speedup vs baseline: 1.2143x; 1.0003x over previous
import functools

import jax
import jax.numpy as jnp
from jax import lax
from jax.experimental import pallas as pl
from jax.experimental.pallas import tpu as pltpu
from jax.experimental.pallas import tpu_sc as plsc

NC = 2
NS = 16
NW = NC * NS
K = 80
NBUF = 5
DEG_W = 16

@functools.cache
def _mesh():
    return plsc.VectorSubcoreMesh(core_axis_name="c", subcore_axis_name="s")


_SC_PARAMS = pltpu.CompilerParams(use_tc_tiling_on_sc=False)
_TC_PARAMS = pltpu.CompilerParams(dimension_semantics=("parallel",))


def _writeback(acc_sh, out_hbm, cid, sid, rps, width, sem):
    nb = rps // 8

    @pl.loop(0, nb)
    def _(g):
        pltpu.async_copy(
            acc_sh.at[pl.ds(sid * rps + g * 8, 8)],
            out_hbm.at[cid, sid * nb + g, :, pl.ds(0, width)],
            sem)

    @pl.loop(0, nb)
    def _(g):
        pltpu.make_async_copy(
            acc_sh.at[pl.ds(sid * rps + g * 8, 8)],
            out_hbm.at[cid, sid * nb + g, :, pl.ds(0, width)],
            sem).wait()


def _zero_spmem(zbuf, acc_sh, sid, rows_per_sub, zr, width):
    @pl.loop(0, zr)
    def _(i):
        for j in range(width // 16):
            zbuf[i, pl.ds(j * 16, 16)] = jnp.zeros((16,), jnp.float32)

    for r in range(rows_per_sub // zr):
        pltpu.sync_copy(zbuf, acc_sh.at[pl.ds(sid * rows_per_sub + r * zr, zr)])


def _pad_rows(n):
    unit = NS * 8 * 16
    return -(-n // unit) * unit


def _make_deg_call(n, e):
    epw = e // NW
    chunks = epw // K
    npad = _pad_rows(n)
    rps = npad // NS
    zr = min(rps, 128)
    assert rps % zr == 0

    @functools.partial(
        pl.kernel,
        mesh=_mesh(),
        out_type=jax.ShapeDtypeStruct((NC, npad // 8, 8, 128), jnp.float32),
        scratch_types=[
            pltpu.VMEM((chunks, K), jnp.int32),
            pltpu.VMEM((K, DEG_W), jnp.float32),
            pltpu.VMEM((zr, DEG_W), jnp.float32),
            pltpu.VMEM_SHARED((npad, DEG_W), jnp.float32),
            pltpu.SemaphoreType.DMA,
            pltpu.SemaphoreType.DMA,
        ],
        compiler_params=_SC_PARAMS,
    )
    def deg_call(ei_hbm, out_hbm, idx_v, ones_v, zbuf, acc_sh, sem, ssem):
        cid = lax.axis_index("c")
        sid = lax.axis_index("s")
        wid = sid * NC + cid

        @pl.loop(0, K)
        def _(i):
            ones_v[i, pl.ds(0, 16)] = jnp.full((16,), 1.0, jnp.float32)

        pltpu.async_copy(ei_hbm.at[1, wid], idx_v, sem)
        _zero_spmem(zbuf, acc_sh, sid, rps, zr, DEG_W)
        pltpu.make_async_copy(ei_hbm.at[1, wid], idx_v, sem).wait()
        plsc.subcore_barrier()

        @pl.loop(0, chunks)
        def _(c):
            pltpu.async_copy(ones_v, acc_sh.at[idx_v.at[c]], ssem, add=True)

        @pl.loop(0, chunks)
        def _(c):
            pltpu.make_async_copy(
                ones_v, acc_sh.at[idx_v.at[c]], ssem).wait()

        plsc.subcore_barrier()
        _writeback(acc_sh, out_hbm, cid, sid, rps, DEG_W, sem)

    return deg_call


def _make_edge_call(n, e, hid):
    epw = e // NW
    chunks = epw // K
    npad = _pad_rows(n)
    rps = npad // NS
    zr = min(rps, 128)
    assert rps % zr == 0

    @functools.partial(
        pl.kernel,
        mesh=_mesh(),
        out_type=jax.ShapeDtypeStruct((NC, npad // 8, 8, 128), jnp.float32),
        scratch_types=[
            pltpu.VMEM((chunks, K), jnp.int32),
            pltpu.VMEM((chunks, K), jnp.int32),
            [pltpu.VMEM((K, hid), jnp.float32) for _ in range(NBUF)],
            pltpu.VMEM((zr, hid), jnp.float32),
            pltpu.VMEM_SHARED((npad, hid), jnp.float32),
            [pltpu.SemaphoreType.DMA for _ in range(NBUF)],
            pltpu.SemaphoreType.DMA,
        ],
        compiler_params=_SC_PARAMS,
    )
    def edge_call(hs_hbm, ei_hbm, out_hbm,
                  sidx_v, didx_v, rows, zbuf, acc_sh, gsems, isem):
        cid = lax.axis_index("c")
        sid = lax.axis_index("s")
        wid = sid * NC + cid

        pltpu.async_copy(ei_hbm.at[0, wid], sidx_v, isem)
        pltpu.async_copy(ei_hbm.at[1, wid], didx_v, isem)
        _zero_spmem(zbuf, acc_sh, sid, rps, zr, hid)
        pltpu.make_async_copy(ei_hbm.at[0, wid], sidx_v, isem).wait()
        pltpu.make_async_copy(ei_hbm.at[1, wid], didx_v, isem).wait()

        for b in range(NBUF):
            pltpu.async_copy(hs_hbm.at[sidx_v.at[b]], rows[b], gsems[b])

        plsc.subcore_barrier()

        @pl.loop(0, chunks, step=NBUF)
        def _(g):
            for b in range(NBUF):
                c = g + b
                pltpu.make_async_copy(
                    hs_hbm.at[sidx_v.at[c]], rows[b], gsems[b]).wait()
                pltpu.sync_copy(rows[b], acc_sh.at[didx_v.at[c]], add=True)
                nxt = c + NBUF

                @pl.when(nxt < chunks)
                def _():
                    pltpu.async_copy(
                        hs_hbm.at[sidx_v.at[nxt]], rows[b], gsems[b])

        plsc.subcore_barrier()
        _writeback(acc_sh, out_hbm, cid, sid, rps, hid, isem)

    return edge_call


def _deg_dis(dacc_ref, rb):
    d = jnp.reshape(dacc_ref[:, :, :, 0:1], (2, rb, 1))
    deg = d[0] + d[1] + 1.0
    return lax.rsqrt(deg)


def _unpack_acc(acc_ref, rb, hid):
    a = jnp.reshape(acc_ref[:, :, :, 0:hid], (2, rb, hid))
    return a[0] + a[1]


def _kmm_body(x_ref, w1_ref, h_ref):
    h_ref[...] = jnp.dot(x_ref[...], w1_ref[...],
                         preferred_element_type=jnp.float32)


def _ksc_body(dacc_ref, h_ref, hs_ref):
    dis = _deg_dis(dacc_ref, h_ref.shape[0])
    hs_ref[...] = h_ref[...] * dis


def _k2_body(dacc_ref, acc_ref, hs1_ref, w2_ref, b1_ref, h2s_ref):
    rb, hid = hs1_ref.shape
    dis = _deg_dis(dacc_ref, rb)
    z = dis * (_unpack_acc(acc_ref, rb, hid) + hs1_ref[...]) + b1_ref[...]
    a = jnp.where(z >= 0, z, 0.01 * z)
    h2 = jnp.dot(a, w2_ref[...], preferred_element_type=jnp.float32)
    h2s_ref[...] = h2 * dis


def _k3_body(dacc_ref, acc_ref, hs2_ref, b2_ref, out_ref):
    rb, hid = hs2_ref.shape
    dis = _deg_dis(dacc_ref, rb)
    z = dis * (_unpack_acc(acc_ref, rb, hid) + hs2_ref[...]) + b2_ref[...]
    out_ref[...] = jnp.where(z >= 0, z, 0.01 * z)


def kernel(x, edge_index, W1, b1, W2, b2):
    n, in_ch = x.shape
    e = edge_index.shape[1]
    hid = W1.shape[1]
    npad = _pad_rows(n)
    rb = 1024
    grid = (npad // rb,)

    epw = e // NW
    chunks = epw // K
    assert epw % K == 0 and chunks % NBUF == 0
    ei = edge_index.astype(jnp.int32).reshape(2, NW, chunks, K)

    deg_call = _make_deg_call(n, e)
    edge_call = _make_edge_call(n, e, hid)

    dacc = deg_call(ei)

    deg_spec = pl.BlockSpec((NC, rb // 8, 8, 128), lambda i: (0, i, 0, 0))
    accs_spec = pl.BlockSpec((NC, rb // 8, 8, 128), lambda i: (0, i, 0, 0))
    row_spec = pl.BlockSpec((rb, hid), lambda i: (i, 0))
    bias_spec = pl.BlockSpec((1, hid), lambda i: (0, 0))

    h1 = pl.pallas_call(
        _kmm_body,
        grid=grid,
        in_specs=[
            pl.BlockSpec((rb, in_ch), lambda i: (i, 0)),
            pl.BlockSpec((in_ch, hid), lambda i: (0, 0)),
        ],
        out_specs=row_spec,
        out_shape=jax.ShapeDtypeStruct((npad, hid), jnp.float32),
        compiler_params=_TC_PARAMS,
    )(x, W1)

    h1s = pl.pallas_call(
        _ksc_body,
        grid=grid,
        in_specs=[deg_spec, row_spec],
        out_specs=row_spec,
        out_shape=jax.ShapeDtypeStruct((npad, hid), jnp.float32),
        compiler_params=_TC_PARAMS,
    )(dacc, h1)

    acc1 = edge_call(h1s, ei)

    h2s = pl.pallas_call(
        _k2_body,
        grid=grid,
        in_specs=[
            deg_spec,
            accs_spec,
            row_spec,
            pl.BlockSpec((hid, hid), lambda i: (0, 0)),
            bias_spec,
        ],
        out_specs=row_spec,
        out_shape=jax.ShapeDtypeStruct((npad, hid), jnp.float32),
        compiler_params=_TC_PARAMS,
    )(dacc, acc1, h1s, W2, b1.reshape(1, hid))

    acc2 = edge_call(h2s, ei)

    out = pl.pallas_call(
        _k3_body,
        grid=grid,
        in_specs=[deg_spec, accs_spec, row_spec, bias_spec],
        out_specs=row_spec,
        out_shape=jax.ShapeDtypeStruct((n, hid), jnp.float32),
        compiler_params=_TC_PARAMS,
    )(dacc, acc2, h2s, b2.reshape(1, hid))

    return out


# speedup vs baseline: 1.2233x; 1.0074x over previous
import functools

import jax
import jax.numpy as jnp
from jax import lax
from jax.experimental import pallas as pl
from jax.experimental.pallas import tpu as pltpu
from jax.experimental.pallas import tpu_sc as plsc

NC = 2
NS = 16
NW = NC * NS
K = 80
NBUF = 5
DEG_W = 16

@functools.cache
def _mesh():
    return plsc.VectorSubcoreMesh(core_axis_name="c", subcore_axis_name="s")


_SC_PARAMS = pltpu.CompilerParams(use_tc_tiling_on_sc=False)
_TC_PARAMS = pltpu.CompilerParams(dimension_semantics=("parallel",))


def _writeback(acc_sh, out_hbm, cid, sid, rps, width, sem):
    nb = rps // 8

    @pl.loop(0, nb)
    def _(g):
        pltpu.async_copy(
            acc_sh.at[pl.ds(sid * rps + g * 8, 8)],
            out_hbm.at[cid, sid * nb + g, :, pl.ds(0, width)],
            sem)

    @pl.loop(0, nb)
    def _(g):
        pltpu.make_async_copy(
            acc_sh.at[pl.ds(sid * rps + g * 8, 8)],
            out_hbm.at[cid, sid * nb + g, :, pl.ds(0, width)],
            sem).wait()


def _zero_spmem(zbuf, acc_sh, sid, rows_per_sub, zr, width):
    @pl.loop(0, zr)
    def _(i):
        for j in range(width // 16):
            zbuf[i, pl.ds(j * 16, 16)] = jnp.zeros((16,), jnp.float32)

    for r in range(rows_per_sub // zr):
        pltpu.sync_copy(zbuf, acc_sh.at[pl.ds(sid * rows_per_sub + r * zr, zr)])


def _pad_rows(n):
    unit = NS * 8 * 16
    return -(-n // unit) * unit


def _make_deg_call(n, e):
    epw = e // NW
    chunks = epw // K
    npad = _pad_rows(n)
    rps = npad // NS
    zr = min(rps, 128)
    assert rps % zr == 0

    @functools.partial(
        pl.kernel,
        mesh=_mesh(),
        out_type=jax.ShapeDtypeStruct((NC, npad // 8, 8, 128), jnp.float32),
        scratch_types=[
            pltpu.VMEM((chunks, K), jnp.int32),
            pltpu.VMEM((K, DEG_W), jnp.float32),
            pltpu.VMEM((zr, DEG_W), jnp.float32),
            pltpu.VMEM_SHARED((npad, DEG_W), jnp.float32),
            pltpu.SemaphoreType.DMA,
            pltpu.SemaphoreType.DMA,
        ],
        compiler_params=_SC_PARAMS,
    )
    def deg_call(ei_hbm, out_hbm, idx_v, ones_v, zbuf, acc_sh, sem, ssem):
        cid = lax.axis_index("c")
        sid = lax.axis_index("s")
        wid = sid * NC + cid

        @pl.loop(0, K)
        def _(i):
            ones_v[i, pl.ds(0, 16)] = jnp.full((16,), 1.0, jnp.float32)

        pltpu.async_copy(ei_hbm.at[1, wid], idx_v, sem)
        _zero_spmem(zbuf, acc_sh, sid, rps, zr, DEG_W)
        pltpu.make_async_copy(ei_hbm.at[1, wid], idx_v, sem).wait()
        plsc.subcore_barrier()

        @pl.loop(0, chunks)
        def _(c):
            pltpu.async_copy(ones_v, acc_sh.at[idx_v.at[c]], ssem, add=True)

        @pl.loop(0, chunks)
        def _(c):
            pltpu.make_async_copy(
                ones_v, acc_sh.at[idx_v.at[c]], ssem).wait()

        plsc.subcore_barrier()
        _writeback(acc_sh, out_hbm, cid, sid, rps, DEG_W, sem)

    return deg_call


def _make_edge_call(n, e, hid):
    epw = e // NW
    chunks = epw // K
    npad = _pad_rows(n)
    rps = npad // NS
    zr = min(rps, 128)
    assert rps % zr == 0

    @functools.partial(
        pl.kernel,
        mesh=_mesh(),
        out_type=jax.ShapeDtypeStruct((NC, npad // 8, 8, 128), jnp.float32),
        scratch_types=[
            pltpu.VMEM((chunks, K), jnp.int32),
            pltpu.VMEM((chunks, K), jnp.int32),
            [pltpu.VMEM((K, hid), jnp.float32) for _ in range(NBUF)],
            pltpu.VMEM((zr, hid), jnp.float32),
            pltpu.VMEM_SHARED((npad, hid), jnp.float32),
            [pltpu.SemaphoreType.DMA for _ in range(NBUF)],
            pltpu.SemaphoreType.DMA,
        ],
        compiler_params=_SC_PARAMS,
    )
    def edge_call(hs_hbm, ei_hbm, out_hbm,
                  sidx_v, didx_v, rows, zbuf, acc_sh, gsems, isem):
        cid = lax.axis_index("c")
        sid = lax.axis_index("s")
        wid = sid * NC + cid

        pltpu.async_copy(ei_hbm.at[0, wid], sidx_v, isem)
        pltpu.async_copy(ei_hbm.at[1, wid], didx_v, isem)
        _zero_spmem(zbuf, acc_sh, sid, rps, zr, hid)
        pltpu.make_async_copy(ei_hbm.at[0, wid], sidx_v, isem).wait()
        pltpu.make_async_copy(ei_hbm.at[1, wid], didx_v, isem).wait()

        for b in range(NBUF):
            pltpu.async_copy(hs_hbm.at[sidx_v.at[b]], rows[b], gsems[b])

        plsc.subcore_barrier()

        @pl.loop(0, chunks, step=NBUF)
        def _(g):
            for b in range(NBUF):
                c = g + b
                pltpu.make_async_copy(
                    hs_hbm.at[sidx_v.at[c]], rows[b], gsems[b]).wait()
                pltpu.sync_copy(rows[b], acc_sh.at[didx_v.at[c]], add=True)
                nxt = c + NBUF

                @pl.when(nxt < chunks)
                def _():
                    pltpu.async_copy(
                        hs_hbm.at[sidx_v.at[nxt]], rows[b], gsems[b])

        plsc.subcore_barrier()
        _writeback(acc_sh, out_hbm, cid, sid, rps, hid, isem)

    return edge_call


def _deg_dis(dacc_ref, rb):
    d = jnp.reshape(dacc_ref[:, :, :, 0:1], (2, rb, 1))
    deg = d[0] + d[1] + 1.0
    return lax.rsqrt(deg)


def _unpack_acc(acc_ref, rb, hid):
    a = jnp.reshape(acc_ref[:, :, :, 0:hid], (2, rb, hid))
    return a[0] + a[1]


def _kmm_body(x_ref, w1_ref, h_ref):
    h_ref[...] = jnp.dot(x_ref[...], w1_ref[...],
                         preferred_element_type=jnp.float32)


def _ksc_body(dacc_ref, h_ref, hs_ref, disb_ref):
    rb, hid = h_ref.shape
    dis = _deg_dis(dacc_ref, rb)
    hs_ref[...] = h_ref[...] * dis
    disb_ref[...] = jnp.broadcast_to(dis, (rb, hid))


def _k2_body(disb_ref, acc_ref, hs1_ref, w2_ref, b1_ref, h2s_ref):
    rb, hid = hs1_ref.shape
    dis = disb_ref[...]
    z = dis * (_unpack_acc(acc_ref, rb, hid) + hs1_ref[...]) + b1_ref[...]
    a = jnp.where(z >= 0, z, 0.01 * z)
    h2 = jnp.dot(a, w2_ref[...], preferred_element_type=jnp.float32)
    h2s_ref[...] = h2 * dis


def _k3_body(disb_ref, acc_ref, hs2_ref, b2_ref, out_ref):
    rb, hid = hs2_ref.shape
    dis = disb_ref[...]
    z = dis * (_unpack_acc(acc_ref, rb, hid) + hs2_ref[...]) + b2_ref[...]
    out_ref[...] = jnp.where(z >= 0, z, 0.01 * z)


def kernel(x, edge_index, W1, b1, W2, b2):
    n, in_ch = x.shape
    e = edge_index.shape[1]
    hid = W1.shape[1]
    npad = _pad_rows(n)
    rb = 1024
    grid = (npad // rb,)

    epw = e // NW
    chunks = epw // K
    assert epw % K == 0 and chunks % NBUF == 0
    ei = edge_index.astype(jnp.int32).reshape(2, NW, chunks, K)

    deg_call = _make_deg_call(n, e)
    edge_call = _make_edge_call(n, e, hid)

    dacc = deg_call(ei)

    deg_spec = pl.BlockSpec((NC, rb // 8, 8, 128), lambda i: (0, i, 0, 0))
    accs_spec = pl.BlockSpec((NC, rb // 8, 8, 128), lambda i: (0, i, 0, 0))
    row_spec = pl.BlockSpec((rb, hid), lambda i: (i, 0))
    bias_spec = pl.BlockSpec((1, hid), lambda i: (0, 0))

    h1 = pl.pallas_call(
        _kmm_body,
        grid=grid,
        in_specs=[
            pl.BlockSpec((rb, in_ch), lambda i: (i, 0)),
            pl.BlockSpec((in_ch, hid), lambda i: (0, 0)),
        ],
        out_specs=row_spec,
        out_shape=jax.ShapeDtypeStruct((npad, hid), jnp.float32),
        compiler_params=_TC_PARAMS,
    )(x, W1)

    h1s, disb = pl.pallas_call(
        _ksc_body,
        grid=grid,
        in_specs=[deg_spec, row_spec],
        out_specs=[row_spec, row_spec],
        out_shape=[
            jax.ShapeDtypeStruct((npad, hid), jnp.float32),
            jax.ShapeDtypeStruct((npad, hid), jnp.float32),
        ],
        compiler_params=_TC_PARAMS,
    )(dacc, h1)

    acc1 = edge_call(h1s, ei)

    h2s = pl.pallas_call(
        _k2_body,
        grid=grid,
        in_specs=[
            row_spec,
            accs_spec,
            row_spec,
            pl.BlockSpec((hid, hid), lambda i: (0, 0)),
            bias_spec,
        ],
        out_specs=row_spec,
        out_shape=jax.ShapeDtypeStruct((npad, hid), jnp.float32),
        compiler_params=_TC_PARAMS,
    )(disb, acc1, h1s, W2, b1.reshape(1, hid))

    acc2 = edge_call(h2s, ei)

    out = pl.pallas_call(
        _k3_body,
        grid=grid,
        in_specs=[row_spec, accs_spec, row_spec, bias_spec],
        out_specs=row_spec,
        out_shape=jax.ShapeDtypeStruct((n, hid), jnp.float32),
        compiler_params=_TC_PARAMS,
    )(disb, acc2, h2s, b2.reshape(1, hid))

    return out
